# baseline (speedup 1.0000x reference)
import functools

import jax
import jax.numpy as jnp
from jax import lax
from jax.experimental import pallas as pl
from jax.experimental.pallas import tpu as pltpu
from jax.experimental.pallas import tpu_sc as plsc

_NC = 2
_NS = 16
_NW = _NC * _NS
_L = 16
_ROWS = 8192
_COLS = 1024
_CHR = 8
_NBUF = 4


def _sc_quant_body(x_hbm, c1_hbm, c2_hbm, o_hbm, *refs):
    xbs = refs[0:_NBUF]
    obs = refs[_NBUF:2 * _NBUF]
    c1buf, c2buf = refs[2 * _NBUF:2 * _NBUF + 2]
    sis = refs[2 * _NBUF + 2:3 * _NBUF + 2]
    sos = refs[3 * _NBUF + 2:4 * _NBUF + 2]

    wid = lax.axis_index("s") * _NC + lax.axis_index("c")
    per_w = _ROWS // _NW
    base = wid * per_w
    nch = per_w // _CHR

    pltpu.sync_copy(c1_hbm, c1buf)
    pltpu.sync_copy(c2_hbm, c2buf)
    c1 = c1buf[...]
    c2 = c2buf[...]

    def compute(xb, ob):
        for r in range(0, _CHR, 4):
            @plsc.parallel_loop(0, _COLS, step=_L, unroll=8)
            def _vec(i):
                for rr in (r, r + 1, r + 2, r + 3):
                    t = xb[rr, pl.ds(i, _L)] * c1
                    t = jnp.minimum(jnp.maximum(t, -127.0), 127.0)
                    k = (t + 12582912.0) - 12582912.0
                    ob[rr, pl.ds(i, _L)] = k * c2

    for b in range(_NBUF):
        pltpu.async_copy(x_hbm.at[pl.ds(base + b * _CHR, _CHR)], xbs[b], sis[b])

    @pl.loop(0, nch, step=_NBUF)
    def _group(c):
        for b in range(_NBUF):
            xb, ob, si, so = xbs[b], obs[b], sis[b], sos[b]
            cc = c + b
            pltpu.make_async_copy(x_hbm.at[pl.ds(base, _CHR)], xb, si).wait()

            @pl.when(cc >= _NBUF)
            def _wait_out():
                pltpu.make_async_copy(ob, o_hbm.at[pl.ds(base, _CHR)], so).wait()

            compute(xb, ob)
            pltpu.async_copy(ob, o_hbm.at[pl.ds(base + cc * _CHR, _CHR)], so)

            @pl.when(cc + _NBUF < nch)
            def _next_in():
                pltpu.async_copy(
                    x_hbm.at[pl.ds(base + (cc + _NBUF) * _CHR, _CHR)], xb, si)

    for b in range(_NBUF):
        pltpu.make_async_copy(obs[b], o_hbm.at[pl.ds(base, _CHR)], sos[b]).wait()


def kernel(x, quant_grid, alpha):
    maxval = jnp.max(quant_grid)
    n_levels = quant_grid.shape[0]
    step = (maxval - jnp.min(quant_grid)) / jnp.float32(n_levels - 1)
    c1 = (maxval / (alpha * step)).astype(jnp.float32)
    c2 = (step * alpha / maxval).astype(jnp.float32)
    c1v = jnp.full((_L,), c1, dtype=jnp.float32)
    c2v = jnp.full((_L,), c2, dtype=jnp.float32)

    xf = x.reshape(_ROWS, _COLS)

    scratch = (
        [pltpu.VMEM((_CHR, _COLS), jnp.float32) for _ in range(2 * _NBUF)]
        + [pltpu.VMEM((_L,), jnp.float32) for _ in range(2)]
        + [pltpu.SemaphoreType.DMA for _ in range(2 * _NBUF)]
    )

    run = functools.partial(
        pl.kernel,
        out_type=jax.ShapeDtypeStruct((_ROWS, _COLS), jnp.float32),
        mesh=plsc.VectorSubcoreMesh(core_axis_name="c", subcore_axis_name="s"),
        scratch_types=scratch,
    )(_sc_quant_body)
    out = run(xf, c1v, c2v)
    return out.reshape(x.shape)


# speedup vs baseline: 1.1427x; 1.1427x over previous
import functools

import jax
import jax.numpy as jnp
from jax import lax
from jax.experimental import pallas as pl
from jax.experimental.pallas import tpu as pltpu
from jax.experimental.pallas import tpu_sc as plsc

_NC = 2
_NS = 16
_NW = _NC * _NS
_L = 16
_ROWS = 8192
_COLS = 1024
_CHR = 8
_NBUF = 4


def _sc_quant_body(x_hbm, c1_hbm, c2_hbm, o_hbm, *refs):
    xbs = refs[0:_NBUF]
    obs = refs[_NBUF:2 * _NBUF]
    c1buf, c2buf = refs[2 * _NBUF:2 * _NBUF + 2]
    sis = refs[2 * _NBUF + 2:3 * _NBUF + 2]
    sos = refs[3 * _NBUF + 2:4 * _NBUF + 2]

    wid = lax.axis_index("s") * _NC + lax.axis_index("c")
    per_w = _ROWS // _NW
    base = wid * per_w
    nch = per_w // _CHR

    pltpu.sync_copy(c1_hbm, c1buf)
    pltpu.sync_copy(c2_hbm, c2buf)
    c1 = c1buf[...]
    c2 = c2buf[...]

    def compute(xb, ob):
        for r in range(0, _CHR, 8):
            @plsc.parallel_loop(0, _COLS, step=_L, unroll=2)
            def _vec(i):
                for rr in range(r, r + 8):
                    t = xb[rr, pl.ds(i, _L)] * c1
                    t = jnp.minimum(jnp.maximum(t, -127.0), 127.0)
                    k = (t + 12582912.0) - 12582912.0
                    ob[rr, pl.ds(i, _L)] = k * c2

    for b in range(_NBUF):
        pltpu.async_copy(x_hbm.at[pl.ds(base + b * _CHR, _CHR)], xbs[b], sis[b])

    @pl.loop(0, nch, step=_NBUF)
    def _group(c):
        for b in range(_NBUF):
            xb, ob, si, so = xbs[b], obs[b], sis[b], sos[b]
            cc = c + b
            pltpu.make_async_copy(x_hbm.at[pl.ds(base, _CHR)], xb, si).wait()

            @pl.when(cc >= _NBUF)
            def _wait_out():
                pltpu.make_async_copy(ob, o_hbm.at[pl.ds(base, _CHR)], so).wait()

            compute(xb, ob)
            pltpu.async_copy(ob, o_hbm.at[pl.ds(base + cc * _CHR, _CHR)], so)

            @pl.when(cc + _NBUF < nch)
            def _next_in():
                pltpu.async_copy(
                    x_hbm.at[pl.ds(base + (cc + _NBUF) * _CHR, _CHR)], xb, si)

    for b in range(_NBUF):
        pltpu.make_async_copy(obs[b], o_hbm.at[pl.ds(base, _CHR)], sos[b]).wait()


def kernel(x, quant_grid, alpha):
    maxval = jnp.max(quant_grid)
    n_levels = quant_grid.shape[0]
    step = (maxval - jnp.min(quant_grid)) / jnp.float32(n_levels - 1)
    c1 = (maxval / (alpha * step)).astype(jnp.float32)
    c2 = (step * alpha / maxval).astype(jnp.float32)
    c1v = jnp.full((_L,), c1, dtype=jnp.float32)
    c2v = jnp.full((_L,), c2, dtype=jnp.float32)

    xf = x.reshape(_ROWS, _COLS)

    scratch = (
        [pltpu.VMEM((_CHR, _COLS), jnp.float32) for _ in range(2 * _NBUF)]
        + [pltpu.VMEM((_L,), jnp.float32) for _ in range(2)]
        + [pltpu.SemaphoreType.DMA for _ in range(2 * _NBUF)]
    )

    run = functools.partial(
        pl.kernel,
        out_type=jax.ShapeDtypeStruct((_ROWS, _COLS), jnp.float32),
        mesh=plsc.VectorSubcoreMesh(core_axis_name="c", subcore_axis_name="s"),
        scratch_types=scratch,
    )(_sc_quant_body)
    out = run(xf, c1v, c2v)
    return out.reshape(x.shape)
